# baseline (speedup 1.0000x reference)
import functools

import jax
import jax.numpy as jnp
from jax import lax
from jax.experimental import pallas as pl
from jax.experimental.pallas import tpu as pltpu
from jax.experimental.pallas import tpu_sc as plsc

B, N, D = 1024, 50, 512
NUM_TYPES = 4
N_PAD = 56

CH = 16
NBUF = 16
LEAD = 8
T = B // CH


def _make_sc_gather():
    mesh = plsc.VectorSubcoreMesh(core_axis_name="c", subcore_axis_name="s")

    @functools.partial(
        pl.kernel,
        mesh=mesh,
        out_type=jax.ShapeDtypeStruct((N_PAD, D), jnp.float32),
        scratch_types=[
            pltpu.VMEM((N_PAD,), jnp.int32),
            pltpu.VMEM((N_PAD, D), jnp.float32),
            pltpu.SemaphoreType.DMA,
        ],
    )
    def gather(tid_hbm, emb_hbm, temb_hbm, tid_v, temb_v, sem):
        wid = lax.axis_index("s") * 2 + lax.axis_index("c")

        @pl.when(wid == 0)
        def _():
            pltpu.sync_copy(tid_hbm, tid_v)
            pltpu.async_copy(emb_hbm.at[tid_v], temb_v, sem).wait()
            pltpu.sync_copy(temb_v, temb_hbm)

    return gather


_sc_gather_cache = []


def _sc_gather(tid, emb):
    if not _sc_gather_cache:
        _sc_gather_cache.append(_make_sc_gather())
    return _sc_gather_cache[0](tid, emb)


def _tc_body(temb_ref, x_hbm, o_hbm, *rest):
    bufs = rest[:NBUF]
    isems = rest[NBUF:2 * NBUF]
    osems = rest[2 * NBUF:3 * NBUF]

    def in_copy(t, p):
        return pltpu.make_async_copy(
            x_hbm.at[pl.ds(t * CH, CH)], bufs[p], isems[p])

    def out_copy(t, p):
        return pltpu.make_async_copy(
            bufs[p], o_hbm.at[pl.ds(t * CH, CH)], osems[p])

    for t in range(LEAD):
        in_copy(t, t % NBUF).start()

    temb = temb_ref[...]

    def step(t0, carry):
        for p in range(NBUF):
            t = t0 + p
            in_copy(t, p).wait()

            @pl.when(t + LEAD < T)
            def _():
                pf = (p + LEAD) % NBUF

                @pl.when(t >= NBUF - LEAD)
                def _():
                    out_copy(t - (NBUF - LEAD), pf).wait()

                in_copy(t + LEAD, pf).start()

            bufs[p][...] = bufs[p][...] + temb[None]
            out_copy(t, p).start()
        return carry

    lax.fori_loop(0, T // NBUF, lambda s, c: step(s * NBUF, c), 0)

    for t in range(T - NBUF, T):
        out_copy(t, t % NBUF).wait()


def kernel(channel_stack, type_ids, embeddings):
    tid = jnp.zeros((N_PAD,), jnp.int32).at[:N].set(type_ids.astype(jnp.int32))
    temb = _sc_gather(tid, embeddings)[:N]
    return pl.pallas_call(
        _tc_body,
        in_specs=[
            pl.BlockSpec(memory_space=pltpu.MemorySpace.VMEM),
            pl.BlockSpec(memory_space=pltpu.MemorySpace.HBM),
        ],
        out_specs=pl.BlockSpec(memory_space=pltpu.MemorySpace.HBM),
        out_shape=jax.ShapeDtypeStruct((B, N, D), jnp.float32),
        scratch_shapes=[pltpu.VMEM((CH, N, D), jnp.float32) for _ in range(NBUF)]
        + [pltpu.SemaphoreType.DMA for _ in range(2 * NBUF)],
    )(temb, channel_stack)
